# baseline (speedup 1.0000x reference)
import functools

import jax
import jax.numpy as jnp
from jax import lax
from jax.experimental import pallas as pl
from jax.experimental.pallas import tpu as pltpu
from jax.experimental.pallas import tpu_sc as plsc

N_EMBED = 2048
NUM_EXPERTS = 64
TOP_K = 8
T_TOKENS = 16384
BLOCK_R = 1024
W_COLS = 256

N_WORKERS = 32
N_CHUNKS = 1
T_CHUNK = T_TOKENS // N_CHUNKS
TOK_PER_WORKER = T_CHUNK // N_WORKERS
LANES = 16


def _proj_body(x_ref, w_ref, b_ref, eps_ref, noisy_ref, skip_ref):
    z = jnp.dot(x_ref[:], w_ref[:], preferred_element_type=jnp.float32)
    z = z + b_ref[:]
    logits = z[:, :NUM_EXPERTS]
    noise_logits = z[:, NUM_EXPERTS:2 * NUM_EXPERTS]
    skip_logit = z[:, 2 * NUM_EXPERTS:2 * NUM_EXPERTS + 1]
    noisy_ref[:] = logits + eps_ref[:] * jax.nn.softplus(noise_logits)
    skip_ref[:] = jax.nn.sigmoid(skip_logit)


def _merge_desc(ka, va, kb, vb):
    kbr = lax.rev(kb, (0,))
    vbr = lax.rev(vb, (0,))
    take_a = ka >= kbr
    kk = jnp.where(take_a, ka, kbr)
    vv = jnp.where(take_a, va, vbr)
    return plsc.sort_key_val(kk, vv, descending=True)


def _sc_route_body(noisy_hbm, router_hbm, idx_hbm, nv, rv, iv):
    wid = lax.axis_index("s") * 2 + lax.axis_index("c")
    base = wid * TOK_PER_WORKER
    pltpu.sync_copy(noisy_hbm.at[pl.ds(base, TOK_PER_WORKER)], nv)

    lane = lax.broadcasted_iota(jnp.int32, (LANES,), 0)
    top8 = lane < TOP_K
    zeros16 = jnp.zeros((LANES,), jnp.float32)

    def route_one(t):
        runs = []
        for j in range(NUM_EXPERTS // LANES):
            vals = nv[t, pl.ds(j * LANES, LANES)]
            runs.append(plsc.sort_key_val(vals, lane + j * LANES,
                                          descending=True))
        k01, v01 = _merge_desc(runs[0][0], runs[0][1], runs[1][0], runs[1][1])
        k23, v23 = _merge_desc(runs[2][0], runs[2][1], runs[3][0], runs[3][1])
        kf, vf = _merge_desc(k01, v01, k23, v23)

        e = jnp.where(top8, jnp.exp(kf - jnp.max(kf)), 0.0)
        p = e / jnp.sum(e)
        for j in range(NUM_EXPERTS // LANES):
            rv[t, pl.ds(j * LANES, LANES)] = zeros16
        tvec = jnp.zeros((LANES,), jnp.int32) + t
        plsc.store_scatter(rv, [tvec, vf], p, mask=top8)
        iv[t] = vf

    def body(i, carry):
        for u in range(4):
            route_one(4 * i + u)
        return carry

    lax.fori_loop(0, TOK_PER_WORKER // 4, body, 0)

    pltpu.sync_copy(rv, router_hbm.at[pl.ds(base, TOK_PER_WORKER)])
    pltpu.sync_copy(iv, idx_hbm.at[pl.ds(base, TOK_PER_WORKER)])


_sc_route = functools.partial(
    pl.kernel,
    out_type=[
        jax.ShapeDtypeStruct((T_CHUNK, NUM_EXPERTS), jnp.float32),
        jax.ShapeDtypeStruct((T_CHUNK, LANES), jnp.int32),
    ],
    mesh=plsc.VectorSubcoreMesh(core_axis_name="c", subcore_axis_name="s"),
    compiler_params=pltpu.CompilerParams(needs_layout_passes=False, use_tc_tiling_on_sc=False),
    scratch_types=[
        pltpu.VMEM((TOK_PER_WORKER, NUM_EXPERTS), jnp.float32),
        pltpu.VMEM((TOK_PER_WORKER, NUM_EXPERTS), jnp.float32),
        pltpu.VMEM((TOK_PER_WORKER, LANES), jnp.int32),
    ],
)(_sc_route_body)


@functools.partial(jax.jit, static_argnames=("interpret",))
def kernel(x, Wr, br, Wn, bn, Ws, bs, eps, interpret=False):
    w = jnp.concatenate(
        [Wr, Wn, Ws, jnp.zeros((W_COLS - 2 * NUM_EXPERTS - 1, N_EMBED), jnp.float32)],
        axis=0,
    ).T
    b = jnp.concatenate(
        [br, bn, bs, jnp.zeros((W_COLS - 2 * NUM_EXPERTS - 1,), jnp.float32)]
    )[None, :]

    proj = pl.pallas_call(
        _proj_body,
        grid=(T_CHUNK // BLOCK_R,),
        in_specs=[
            pl.BlockSpec((BLOCK_R, N_EMBED), lambda i: (i, 0)),
            pl.BlockSpec((N_EMBED, W_COLS), lambda i: (0, 0)),
            pl.BlockSpec((1, W_COLS), lambda i: (0, 0)),
            pl.BlockSpec((BLOCK_R, NUM_EXPERTS), lambda i: (i, 0)),
        ],
        out_specs=[
            pl.BlockSpec((BLOCK_R, NUM_EXPERTS), lambda i: (i, 0)),
            pl.BlockSpec((BLOCK_R, 1), lambda i: (i, 0)),
        ],
        out_shape=[
            jax.ShapeDtypeStruct((T_CHUNK, NUM_EXPERTS), jnp.float32),
            jax.ShapeDtypeStruct((T_CHUNK, 1), jnp.float32),
        ],
        interpret=interpret,
    )

    routers, idxs, skips = [], [], []
    for c in range(N_CHUNKS):
        sl = slice(c * T_CHUNK, (c + 1) * T_CHUNK)
        noisy_c, skip_c = proj(x[sl], w, b, eps[sl])
        router_c, idx16_c = _sc_route(noisy_c)
        routers.append(router_c)
        idxs.append(idx16_c[:, :TOP_K])
        skips.append(skip_c)
    return (
        jnp.concatenate(routers, axis=0),
        jnp.concatenate(idxs, axis=0),
        jnp.concatenate(skips, axis=0),
    )


# speedup vs baseline: 1.0081x; 1.0081x over previous
import functools

import jax
import jax.numpy as jnp
from jax import lax
from jax.experimental import pallas as pl
from jax.experimental.pallas import tpu as pltpu
from jax.experimental.pallas import tpu_sc as plsc

N_EMBED = 2048
NUM_EXPERTS = 64
TOP_K = 8
T_TOKENS = 16384
BLOCK_R = 1024
W_COLS = 256

N_WORKERS = 32
TOK_PER_WORKER = T_TOKENS // N_WORKERS
LANES = 16


def _proj_body(x_ref, w_ref, b_ref, eps_ref, noisy_ref, skip_ref):
    z = jnp.dot(x_ref[:], w_ref[:], preferred_element_type=jnp.float32)
    z = z + b_ref[:]
    logits = z[:, :NUM_EXPERTS]
    noise_logits = z[:, NUM_EXPERTS:2 * NUM_EXPERTS]
    skip_logit = z[:, 2 * NUM_EXPERTS:2 * NUM_EXPERTS + 1]
    noisy_ref[:] = logits + eps_ref[:] * jax.nn.softplus(noise_logits)
    skip_ref[:] = jax.nn.sigmoid(skip_logit)


def _merge_desc(ka, va, kb, vb):
    kbr = lax.rev(kb, (0,))
    vbr = lax.rev(vb, (0,))
    take_a = ka >= kbr
    kk = jnp.where(take_a, ka, kbr)
    vv = jnp.where(take_a, va, vbr)
    return plsc.sort_key_val(kk, vv, descending=True)


def _sc_route_body(noisy_hbm, router_hbm, idx_hbm, nv, rv, iv):
    wid = lax.axis_index("s") * 2 + lax.axis_index("c")
    base = wid * TOK_PER_WORKER
    pltpu.sync_copy(noisy_hbm.at[pl.ds(base, TOK_PER_WORKER)], nv)

    lane = lax.broadcasted_iota(jnp.int32, (LANES,), 0)
    top8 = lane < TOP_K
    zeros16 = jnp.zeros((LANES,), jnp.float32)

    def route_one(t):
        runs = []
        for j in range(NUM_EXPERTS // LANES):
            vals = nv[t, pl.ds(j * LANES, LANES)]
            runs.append(plsc.sort_key_val(vals, lane + j * LANES,
                                          descending=True))
        k01, v01 = _merge_desc(runs[0][0], runs[0][1], runs[1][0], runs[1][1])
        k23, v23 = _merge_desc(runs[2][0], runs[2][1], runs[3][0], runs[3][1])
        kf, vf = _merge_desc(k01, v01, k23, v23)

        e = jnp.where(top8, jnp.exp(kf - jnp.max(kf)), 0.0)
        p = e / jnp.sum(e)
        for j in range(NUM_EXPERTS // LANES):
            rv[t, pl.ds(j * LANES, LANES)] = zeros16
        tvec = jnp.zeros((LANES,), jnp.int32) + t
        plsc.store_scatter(rv, [tvec, vf], p, mask=top8)
        iv[t] = vf

    def body(i, carry):
        route_one(2 * i)
        route_one(2 * i + 1)
        return carry

    lax.fori_loop(0, TOK_PER_WORKER // 2, body, 0)

    pltpu.sync_copy(rv, router_hbm.at[pl.ds(base, TOK_PER_WORKER)])
    pltpu.sync_copy(iv, idx_hbm.at[pl.ds(base, TOK_PER_WORKER)])


_sc_route = functools.partial(
    pl.kernel,
    out_type=[
        jax.ShapeDtypeStruct((T_TOKENS, NUM_EXPERTS), jnp.float32),
        jax.ShapeDtypeStruct((T_TOKENS, LANES), jnp.int32),
    ],
    mesh=plsc.VectorSubcoreMesh(core_axis_name="c", subcore_axis_name="s"),
    compiler_params=pltpu.CompilerParams(needs_layout_passes=False, use_tc_tiling_on_sc=False),
    scratch_types=[
        pltpu.VMEM((TOK_PER_WORKER, NUM_EXPERTS), jnp.float32),
        pltpu.VMEM((TOK_PER_WORKER, NUM_EXPERTS), jnp.float32),
        pltpu.VMEM((TOK_PER_WORKER, LANES), jnp.int32),
    ],
)(_sc_route_body)


@functools.partial(jax.jit, static_argnames=("interpret",))
def kernel(x, Wr, br, Wn, bn, Ws, bs, eps, interpret=False):
    w = jnp.concatenate(
        [Wr, Wn, Ws, jnp.zeros((W_COLS - 2 * NUM_EXPERTS - 1, N_EMBED), jnp.float32)],
        axis=0,
    ).T
    b = jnp.concatenate(
        [br, bn, bs, jnp.zeros((W_COLS - 2 * NUM_EXPERTS - 1,), jnp.float32)]
    )[None, :]

    proj = pl.pallas_call(
        _proj_body,
        grid=(T_TOKENS // BLOCK_R,),
        in_specs=[
            pl.BlockSpec((BLOCK_R, N_EMBED), lambda i: (i, 0)),
            pl.BlockSpec((N_EMBED, W_COLS), lambda i: (0, 0)),
            pl.BlockSpec((1, W_COLS), lambda i: (0, 0)),
            pl.BlockSpec((BLOCK_R, NUM_EXPERTS), lambda i: (i, 0)),
        ],
        out_specs=[
            pl.BlockSpec((BLOCK_R, NUM_EXPERTS), lambda i: (i, 0)),
            pl.BlockSpec((BLOCK_R, 1), lambda i: (i, 0)),
        ],
        out_shape=[
            jax.ShapeDtypeStruct((T_TOKENS, NUM_EXPERTS), jnp.float32),
            jax.ShapeDtypeStruct((T_TOKENS, 1), jnp.float32),
        ],
        interpret=interpret,
    )

    noisy, skip = proj(x, w, b, eps)
    router, idx16 = _sc_route(noisy)
    return (router, idx16[:, :TOP_K], skip)
